# Initial kernel scaffold; baseline (speedup 1.0000x reference)
import jax
import jax.numpy as jnp
from jax.experimental import pallas as pl


def kernel(x, edge_attr, feat_mask, params, edge_index, batch, equality):
    raise NotImplementedError("write your pallas kernel here")


# speedup vs baseline: 2.0782x; 2.0782x over previous
import functools

import jax
import jax.numpy as jnp
import numpy as np
from jax import lax
from jax.experimental import pallas as pl
from jax.experimental.pallas import tpu as pltpu
from jax.experimental.pallas import tpu_sc as plsc

N = 10000
E = 160000
B = 128
EMB = 128
BINS = 512
FIN = 92

F32 = jnp.float32


_BE = 640
_BN = 1000


def _edge_feat_body(gamma_ref, ea_ref, centers_ref, wrbf_ref, brbf_ref,
                    wk0_ref, bk0_ref, wm0_ref, bm0_ref,
                    wk1_ref, bk1_ref, wm1_ref, bm1_ref,
                    we_ref, be_ref,
                    ef0_ref, ef1_ref, efe_ref):
    ea = ea_ref[...]
    nrm2 = ea[0] ** 2 + ea[1] ** 2 + ea[2] ** 2
    s = -0.75 * lax.rsqrt(nrm2)
    gamma = gamma_ref[0]
    d = s[:, None] - centers_ref[...]
    r = jnp.exp(-gamma * d * d)
    ef = jax.nn.softplus(
        jnp.dot(r, wrbf_ref[...], preferred_element_type=F32) + brbf_ref[...])
    dotp = lambda w, b: jnp.dot(ef, w[...], preferred_element_type=F32) + b[...]
    ef0_ref[...] = jnp.concatenate([dotp(wk0_ref, bk0_ref),
                                    dotp(wm0_ref, bm0_ref)], axis=1)
    ef1_ref[...] = jnp.concatenate([dotp(wk1_ref, bk1_ref),
                                    dotp(wm1_ref, bm1_ref)], axis=1)
    efe_ref[...] = dotp(we_ref, be_ref)


def _edge_features(ea_t, centers, gamma, p):
    g = E // _BE
    full = lambda shp: pl.BlockSpec(shp, lambda i: (0,) * len(shp))
    a0, a1, eq = p['att0'], p['att1'], p['equi']
    args = (ea_t, centers.reshape(1, BINS), p['W_rbf'], p['b_rbf'].reshape(1, EMB),
            a0['Wku'][2 * EMB:], a0['bku'].reshape(1, EMB),
            a0['Wmu'][2 * EMB:], a0['bmu'].reshape(1, EMB),
            a1['Wku'][2 * EMB:], a1['bku'].reshape(1, EMB),
            a1['Wmu'][2 * EMB:], a1['bmu'].reshape(1, EMB),
            eq['We1'][2 * EMB:], eq['be1'].reshape(1, EMB))
    in_specs = [pl.BlockSpec((3, _BE), lambda i: (0, i)),
                full((1, BINS)), full((BINS, EMB)), full((1, EMB)),
                full((EMB, EMB)), full((1, EMB)), full((EMB, EMB)), full((1, EMB)),
                full((EMB, EMB)), full((1, EMB)), full((EMB, EMB)), full((1, EMB)),
                full((EMB, EMB)), full((1, EMB))]
    return pl.pallas_call(
        _edge_feat_body,
        grid=(g,),
        in_specs=[pl.BlockSpec(memory_space=pltpu.SMEM)] + in_specs,
        out_specs=[pl.BlockSpec((_BE, 2 * EMB), lambda i: (i, 0)),
                   pl.BlockSpec((_BE, 2 * EMB), lambda i: (i, 0)),
                   pl.BlockSpec((_BE, EMB), lambda i: (i, 0))],
        out_shape=[jax.ShapeDtypeStruct((E, 2 * EMB), F32),
                   jax.ShapeDtypeStruct((E, 2 * EMB), F32),
                   jax.ShapeDtypeStruct((E, EMB), F32)],
    )(gamma.reshape(1), *args)


_SCALE = float(1.0 / np.sqrt(float(EMB)))


def _qkv_tables(nf, cp):
    q = (jnp.dot(nf, cp['Wq'], preferred_element_type=F32) + cp['bq']) * _SCALE
    k = jnp.dot(nf, cp['Wk'], preferred_element_type=F32) + cp['bk']
    v = jnp.dot(nf, cp['Wv'], preferred_element_type=F32) + cp['bv']
    ka = jnp.dot(k, cp['WkuA'], preferred_element_type=F32)
    kb = jnp.dot(k, cp['WkuB'], preferred_element_type=F32)
    va = jnp.dot(v, cp['WmuA'], preferred_element_type=F32)
    vb = jnp.dot(v, cp['WmuB'], preferred_element_type=F32)
    tdst = jnp.concatenate([q, ka, va], axis=1)
    tsrc = jnp.concatenate([kb, vb], axis=1)
    return tdst, tsrc


def _conv_param_args(cp):
    return (cp['Wq'], cp['bq'].reshape(1, EMB), cp['Wk'], cp['bk'].reshape(1, EMB),
            cp['Wv'], cp['bv'].reshape(1, EMB),
            cp['Wku'][:EMB], cp['Wku'][EMB:2 * EMB],
            cp['Wmu'][:EMB], cp['Wmu'][EMB:2 * EMB])


def _conv_param_dict(refs):
    (wq, bq, wk, bk, wv, bv, wkua, wkub, wmua, wmub) = refs
    return dict(Wq=wq[...], bq=bq[...], Wk=wk[...], bk=bk[...],
                Wv=wv[...], bv=bv[...], WkuA=wkua[...], WkuB=wkub[...],
                WmuA=wmua[...], WmuB=wmub[...])


def _tables0_body(x_ref, wa_ref, ba_ref, *rest):
    cp = _conv_param_dict(rest[:10])
    nf_ref, tdst_ref, tsrc_ref = rest[10:]
    nf = jnp.dot(x_ref[...], wa_ref[...], preferred_element_type=F32) + ba_ref[...]
    nf_ref[...] = nf
    tdst, tsrc = _qkv_tables(nf, cp)
    tdst_ref[...] = tdst
    tsrc_ref[...] = tsrc


def _tables0(x, p):
    g = N // _BN
    full = lambda shp: pl.BlockSpec(shp, lambda i: (0,) * len(shp))
    cp_args = _conv_param_args(p['att0'])
    in_specs = ([pl.BlockSpec((_BN, FIN), lambda i: (i, 0)),
                 full((FIN, EMB)), full((1, EMB))] +
                [full(a.shape) for a in cp_args])
    return pl.pallas_call(
        _tables0_body,
        grid=(g,),
        in_specs=in_specs,
        out_specs=[pl.BlockSpec((_BN, EMB), lambda i: (i, 0)),
                   pl.BlockSpec((_BN, 3 * EMB), lambda i: (i, 0)),
                   pl.BlockSpec((_BN, 2 * EMB), lambda i: (i, 0))],
        out_shape=[jax.ShapeDtypeStruct((N, EMB), F32),
                   jax.ShapeDtypeStruct((N, 3 * EMB), F32),
                   jax.ShapeDtypeStruct((N, 2 * EMB), F32)],
    )(x, p['W_atom'], p['b_atom'].reshape(1, EMB), *cp_args)


def _node_update(nf_prev, agg, cnt, wml, bml):
    out = jnp.dot(agg, wml, preferred_element_type=F32) + cnt * bml
    return jax.nn.softplus(nf_prev + out)


def _update_tables_body(nf_ref, a0_ref, a1_ref, c0_ref, c1_ref,
                        wml_ref, bml_ref, *rest):
    cp = _conv_param_dict(rest[:10])
    nf_ref_o, tdst_ref, tsrc_ref = rest[10:]
    s = a0_ref[...] + a1_ref[...]
    cnt = c0_ref[...] + c1_ref[...]
    nf = _node_update(nf_ref[...], s, cnt, wml_ref[...], bml_ref[...])
    nf_ref_o[...] = nf
    tdst, tsrc = _qkv_tables(nf, cp)
    tdst_ref[...] = tdst
    tsrc_ref[...] = tsrc


def _update_tables(nf, agg, cnt2, conv_prev, conv_next):
    g = N // _BN
    full = lambda shp: pl.BlockSpec(shp, lambda i: (0,) * len(shp))
    cp_args = _conv_param_args(conv_next)
    row = lambda w: pl.BlockSpec((_BN, w), lambda i: (i, 0))
    in_specs = ([row(EMB), row(EMB), row(EMB), row(1), row(1),
                 full((EMB, EMB)), full((1, EMB))] +
                [full(a.shape) for a in cp_args])
    return pl.pallas_call(
        _update_tables_body,
        grid=(g,),
        in_specs=in_specs,
        out_specs=[row(EMB), row(3 * EMB), row(2 * EMB)],
        out_shape=[jax.ShapeDtypeStruct((N, EMB), F32),
                   jax.ShapeDtypeStruct((N, 3 * EMB), F32),
                   jax.ShapeDtypeStruct((N, 2 * EMB), F32)],
    )(nf, agg[0], agg[1], cnt2[0], cnt2[1],
      conv_prev['Wml'], conv_prev['bml'].reshape(1, EMB), *cp_args)


def _equi_tables_body(nf_ref, a0_ref, a1_ref, c0_ref, c1_ref,
                      wml_ref, bml_ref, wes_ref, wed_ref,
                      nf_ref_o, tsrc_ref, tdst_ref):
    cnt = c0_ref[...] + c1_ref[...]
    nf = _node_update(nf_ref[...], a0_ref[...] + a1_ref[...], cnt,
                      wml_ref[...], bml_ref[...])
    nf_ref_o[...] = nf
    tsrc_ref[...] = jnp.dot(nf, wes_ref[...], preferred_element_type=F32)
    tdst_ref[...] = jnp.dot(nf, wed_ref[...], preferred_element_type=F32)


def _equi_tables(nf, agg, cnt2, conv_prev, ep):
    g = N // _BN
    full = lambda shp: pl.BlockSpec(shp, lambda i: (0,) * len(shp))
    row = lambda w: pl.BlockSpec((_BN, w), lambda i: (i, 0))
    in_specs = [row(EMB), row(EMB), row(EMB), row(1), row(1),
                full((EMB, EMB)), full((1, EMB)),
                full((EMB, EMB)), full((EMB, EMB))]
    return pl.pallas_call(
        _equi_tables_body,
        grid=(g,),
        in_specs=in_specs,
        out_specs=[row(EMB), row(EMB), row(EMB)],
        out_shape=[jax.ShapeDtypeStruct((N, EMB), F32),
                   jax.ShapeDtypeStruct((N, EMB), F32),
                   jax.ShapeDtypeStruct((N, EMB), F32)],
    )(nf, agg[0], agg[1], cnt2[0], cnt2[1],
      conv_prev['Wml'], conv_prev['bml'].reshape(1, EMB),
      ep['We1'][:EMB], ep['We1'][EMB:2 * EMB])


def _equi_pool_body(nf_ref, a0_ref, a1_ref, c0_ref, c1_ref, batch_ref,
                    we2_ref, be2_ref, csum_ref, ccnt_ref):
    cnt = jnp.maximum(c0_ref[...] + c1_ref[...], 1.0)
    agg = (a0_ref[...] + a1_ref[...]) / cnt
    nf3 = jax.nn.softplus(
        nf_ref[...] + jnp.dot(agg, we2_ref[...], preferred_element_type=F32)
        + be2_ref[...])
    gid = lax.broadcasted_iota(jnp.int32, (1, B), 1)
    oh = (batch_ref[...] == gid).astype(F32)
    contrib = lax.dot_general(oh, nf3, (((0,), (0,)), ((), ())),
                              preferred_element_type=F32)
    ones = jnp.ones((nf3.shape[0], 1), F32)
    ccontrib = lax.dot_general(oh, ones, (((0,), (0,)), ((), ())),
                               preferred_element_type=F32)

    @pl.when(pl.program_id(0) == 0)
    def _init():
        csum_ref[...] = contrib
        ccnt_ref[...] = ccontrib

    @pl.when(pl.program_id(0) != 0)
    def _acc():
        csum_ref[...] += contrib
        ccnt_ref[...] += ccontrib


def _equi_pool(nf, agg, cnt2, batch2d, ep):
    g = N // _BN
    full = lambda shp: pl.BlockSpec(shp, lambda i: (0,) * len(shp))
    row = lambda w: pl.BlockSpec((_BN, w), lambda i: (i, 0))
    in_specs = [row(EMB), row(EMB), row(EMB), row(1), row(1), row(1),
                full((EMB, EMB)), full((1, EMB))]
    return pl.pallas_call(
        _equi_pool_body,
        grid=(g,),
        in_specs=in_specs,
        out_specs=[full((B, EMB)), full((B, 1))],
        out_shape=[jax.ShapeDtypeStruct((B, EMB), F32),
                   jax.ShapeDtypeStruct((B, 1), F32)],
    )(nf, agg[0], agg[1], cnt2[0], cnt2[1], batch2d,
      ep['We2'], ep['be2'].reshape(1, EMB))


_PAIRS = [(j, k) for j in range(9) for k in range(j + 1, 9)]


def _head_body(csum_ref, ccnt_ref, fm_ref, wout_ref, bout_ref, eq_ref, out_ref):
    crystal = csum_ref[...] / jnp.maximum(ccnt_ref[...], 1.0)
    fm = fm_ref[...]
    crystal2 = jnp.sum(fm * crystal[:, None, :], axis=2)
    out9 = (jnp.dot(crystal2, wout_ref[...], preferred_element_type=F32)
            + bout_ref[...])
    cols = [out9[:, j:j + 1] for j in range(9)]
    for (j, k) in _PAIRS:
        m = eq_ref[:, 9 * j + k:9 * j + k + 1] != 0
        avg = 0.5 * (cols[j] + cols[k])
        cols[j] = jnp.where(m, avg, cols[j])
        cols[k] = jnp.where(m, avg, cols[k])
    out_ref[...] = jnp.concatenate(cols, axis=1)


def _head(csum, ccnt, feat_mask, wout, bout, eqflat):
    full = lambda shp: pl.BlockSpec(shp, lambda: (0,) * len(shp))
    return pl.pallas_call(
        _head_body,
        in_specs=[full((B, EMB)), full((B, 1)), full((B, EMB, EMB)),
                  full((EMB, 9)), full((1, 9)), full((B, 81))],
        out_specs=full((B, 9)),
        out_shape=jax.ShapeDtypeStruct((B, 9), F32),
    )(csum, ccnt, feat_mask, wout, bout.reshape(1, 9), eqflat)


_NC = 2
_NS = 16
_NW = _NC * _NS
_C = 40
_EPT = E // _NW
_NCHUNK = _EPT // _C
_NACC = 10240
_RPT = _NACC // _NS
_HR = _NACC // 128

def _sc_mesh():
    return plsc.VectorSubcoreMesh(core_axis_name="c", subcore_axis_name="s",
                                  num_cores=_NC, num_subcores=_NS)


def _vec_rsqrt(t):
    i = lax.bitcast_convert_type(t, jnp.int32)
    y = lax.bitcast_convert_type(0x5F3759DF - (i >> 1), F32)
    for _ in range(3):
        y = y * (1.5 - 0.5 * t * y * y)
    return y


def _sigmoid(x):
    return 1.0 / (1.0 + jnp.exp(-x))


def _zero_vmem(ref, rows, width):
    def body(i, c):
        for f in range(width // 16):
            ref[i, pl.ds(f * 16, 16)] = jnp.zeros((16,), F32)
        return c
    lax.fori_loop(0, rows, body, 0)


def _copy_zero_to_shared(zbuf, shared, row0, nrows, bufrows):
    r = 0
    while r < nrows:
        sz = min(bufrows, nrows - r)
        pltpu.sync_copy(zbuf.at[pl.ds(0, sz)], shared.at[pl.ds(row0 + r, sz)])
        r += sz


def _conv_edge_body(tdst_hbm, tsrc_hbm, ef_hbm, src_hbm, dst_hbm,
                    agg_hbm, idx_s, idx_d, rows_d, rows_s, ef_b, msg,
                    acc_sh, sem1, sem2, sem3):
    core = lax.axis_index("c")
    sub = lax.axis_index("s")
    wid = core * _NS + sub
    base = wid * _EPT

    _zero_vmem(msg, _C, EMB)
    _copy_zero_to_shared(msg, acc_sh, sub * _RPT, _RPT, _C)
    plsc.subcore_barrier()

    def chunk_body(i, carry):
        off = pl.multiple_of(base + i * _C, 8)
        pltpu.sync_copy(dst_hbm.at[pl.ds(off, _C)], idx_d)
        pltpu.sync_copy(src_hbm.at[pl.ds(off, _C)], idx_s)
        cp1 = pltpu.async_copy(tdst_hbm.at[idx_d], rows_d, sem1)
        cp2 = pltpu.async_copy(tsrc_hbm.at[idx_s], rows_s, sem2)
        cp3 = pltpu.async_copy(ef_hbm.at[pl.ds(off, _C)], ef_b, sem3)
        cp1.wait()
        cp2.wait()
        cp3.wait()

        lanes = lax.iota(jnp.int32, 16)

        def edge_body(e, c):
            alpha = []
            vsum = jnp.zeros((16,), F32)
            for f in range(8):
                q = rows_d[e, pl.ds(f * 16, 16)]
                ka = rows_d[e, pl.ds(EMB + f * 16, 16)]
                kb = rows_s[e, pl.ds(f * 16, 16)]
                efk = ef_b[e, pl.ds(f * 16, 16)]
                a = q * (ka + kb + efk)
                alpha.append(a)
                vsum = vsum + a
            for kk in (8, 4, 2, 1):
                vsum = vsum + vsum[lanes ^ kk]
            mean = vsum * (1.0 / EMB)
            vvar = jnp.zeros((16,), F32)
            for f in range(8):
                dv = alpha[f] - mean
                vvar = vvar + dv * dv
            for kk in (8, 4, 2, 1):
                vvar = vvar + vvar[lanes ^ kk]
            var = vvar * (1.0 / EMB)
            rstd = _vec_rsqrt(var + 1e-5)
            for f in range(8):
                g = _sigmoid((alpha[f] - mean) * rstd)
                va = rows_d[e, pl.ds(2 * EMB + f * 16, 16)]
                vb = rows_s[e, pl.ds(EMB + f * 16, 16)]
                efm = ef_b[e, pl.ds(EMB + f * 16, 16)]
                msg[e, pl.ds(f * 16, 16)] = (va + vb + efm) * g
            return c

        lax.fori_loop(0, _C, edge_body, 0)
        pltpu.sync_copy(msg, acc_sh.at[idx_d], add=True)
        return carry

    lax.fori_loop(0, _NCHUNK, chunk_body, 0)
    plsc.subcore_barrier()

    r0 = sub * _RPT
    pltpu.sync_copy(acc_sh.at[pl.ds(r0, _RPT)], agg_hbm.at[core, pl.ds(r0, _RPT)])


def _make_conv_edge():
    out_type = [jax.ShapeDtypeStruct((_NC, _NACC, EMB), F32)]
    scratch = [pltpu.VMEM((_C,), jnp.int32), pltpu.VMEM((_C,), jnp.int32),
               pltpu.VMEM((_C, 3 * EMB), F32), pltpu.VMEM((_C, 2 * EMB), F32),
               pltpu.VMEM((_C, 2 * EMB), F32), pltpu.VMEM((_C, EMB), F32),
               pltpu.VMEM_SHARED((_NACC, EMB), F32)]
    scratch += [pltpu.SemaphoreType.DMA] * 3
    return pl.kernel(_conv_edge_body,
                     out_type=out_type, mesh=_sc_mesh(), scratch_types=scratch)


_CC = 40
_NCHUNK_C = _EPT // _CC


def _count_edge_body(dst_hbm, cnt_hbm, idx_d, ones_b, zbuf, acc_sh, sem1):
    core = lax.axis_index("c")
    sub = lax.axis_index("s")
    wid = core * _NS + sub
    base = wid * _EPT

    _zero_vmem(zbuf, _CC, EMB)
    _zero_vmem(ones_b, _CC, EMB)
    _copy_zero_to_shared(zbuf, acc_sh, sub * _RPT, _RPT, _CC)
    lane0 = jnp.where(lax.iota(jnp.int32, 16) == 0,
                      jnp.float32(1), jnp.float32(0))

    def fill_ones(i, c):
        ones_b[i, pl.ds(0, 16)] = lane0
        return c
    lax.fori_loop(0, _CC, fill_ones, 0)
    plsc.subcore_barrier()

    def chunk_body(i, carry):
        off = pl.multiple_of(base + i * _CC, 8)
        pltpu.sync_copy(dst_hbm.at[pl.ds(off, _CC)], idx_d)
        pltpu.sync_copy(ones_b, acc_sh.at[idx_d], add=True)
        return carry

    lax.fori_loop(0, _NCHUNK_C, chunk_body, 0)
    plsc.subcore_barrier()
    r0 = sub * _RPT
    pltpu.sync_copy(acc_sh.at[pl.ds(r0, _RPT)], cnt_hbm.at[core, pl.ds(r0, _RPT)])


def _make_count_edge():
    scratch = [pltpu.VMEM((_CC,), jnp.int32), pltpu.VMEM((_CC, EMB), F32),
               pltpu.VMEM((_CC, EMB), F32),
               pltpu.VMEM_SHARED((_NACC, EMB), F32),
               pltpu.SemaphoreType.DMA]
    return pl.kernel(_count_edge_body,
                     out_type=[jax.ShapeDtypeStruct((_NC, _NACC, EMB), F32)],
                     mesh=_sc_mesh(), scratch_types=scratch)


def _equi_edge_body(tsrc_hbm, tdst_hbm, ef_hbm, src_hbm, dst_hbm,
                    agg_hbm, idx_s, idx_d, rows_s, rows_d, ef_b, msg,
                    acc_sh, sem1, sem2, sem3):
    core = lax.axis_index("c")
    sub = lax.axis_index("s")
    wid = core * _NS + sub
    base = wid * _EPT

    _zero_vmem(msg, _C, EMB)
    _copy_zero_to_shared(msg, acc_sh, sub * _RPT, _RPT, _C)
    plsc.subcore_barrier()

    def chunk_body(i, carry):
        off = pl.multiple_of(base + i * _C, 8)
        pltpu.sync_copy(dst_hbm.at[pl.ds(off, _C)], idx_d)
        pltpu.sync_copy(src_hbm.at[pl.ds(off, _C)], idx_s)
        cp1 = pltpu.async_copy(tdst_hbm.at[idx_d], rows_d, sem1)
        cp2 = pltpu.async_copy(tsrc_hbm.at[idx_s], rows_s, sem2)
        cp3 = pltpu.async_copy(ef_hbm.at[pl.ds(off, _C)], ef_b, sem3)
        cp1.wait()
        cp2.wait()
        cp3.wait()

        def edge_body(e, c):
            for f in range(8):
                t = (rows_s[e, pl.ds(f * 16, 16)]
                     + rows_d[e, pl.ds(f * 16, 16)]
                     + ef_b[e, pl.ds(f * 16, 16)])
                msg[e, pl.ds(f * 16, 16)] = t * _sigmoid(t)
            return c

        lax.fori_loop(0, _C, edge_body, 0)
        pltpu.sync_copy(msg, acc_sh.at[idx_d], add=True)
        return carry

    lax.fori_loop(0, _NCHUNK, chunk_body, 0)
    plsc.subcore_barrier()
    r0 = sub * _RPT
    pltpu.sync_copy(acc_sh.at[pl.ds(r0, _RPT)], agg_hbm.at[core, pl.ds(r0, _RPT)])


def _make_equi_edge():
    scratch = [pltpu.VMEM((_C,), jnp.int32), pltpu.VMEM((_C,), jnp.int32),
               pltpu.VMEM((_C, EMB), F32), pltpu.VMEM((_C, EMB), F32),
               pltpu.VMEM((_C, EMB), F32), pltpu.VMEM((_C, EMB), F32),
               pltpu.VMEM_SHARED((_NACC, EMB), F32)]
    scratch += [pltpu.SemaphoreType.DMA] * 3
    return pl.kernel(_equi_edge_body,
                     out_type=[jax.ShapeDtypeStruct((_NC, _NACC, EMB), F32)],
                     mesh=_sc_mesh(), scratch_types=scratch)


def kernel(x, edge_attr, feat_mask, params, edge_index, batch, equality):
    p = params
    src = edge_index[0].astype(jnp.int32)
    dst = edge_index[1].astype(jnp.int32)
    ea_t = edge_attr.T
    centers = jnp.linspace(-4.0, 0.0, BINS)
    gamma = 1.0 / (centers[1] - centers[0]) ** 2

    ef0, ef1, efe = _edge_features(ea_t, centers, gamma, p)
    nf0, td0, ts0 = _tables0(x, p)
    cnts = _make_count_edge()(dst)
    if isinstance(cnts, (tuple, list)):
        cnts = cnts[0]
    cnt2 = cnts[:, :, 0:1]
    agg0 = _make_conv_edge()(td0, ts0, ef0, src, dst)
    if isinstance(agg0, (tuple, list)):
        agg0 = agg0[0]
    nf1, td1, ts1 = _update_tables(nf0, agg0, cnt2, p['att0'], p['att1'])
    agg1 = _make_conv_edge()(td1, ts1, ef1, src, dst)
    if isinstance(agg1, (tuple, list)):
        agg1 = agg1[0]
    nf2, tse, tde = _equi_tables(nf1, agg1, cnt2, p['att1'], p['equi'])
    agge = _make_equi_edge()(tse, tde, efe, src, dst)
    if isinstance(agge, (tuple, list)):
        agge = agge[0]
    csum, ccnt = _equi_pool(nf2, agge, cnt2, batch.astype(jnp.int32).reshape(N, 1),
                            p['equi'])
    eqflat = equality[:, :9, :9].reshape(B, 81).astype(jnp.int32)
    out9 = _head(csum, ccnt, feat_mask, p['W_out'], p['b_out'], eqflat)
    return out9.reshape(B, 3, 3)


# speedup vs baseline: 2.2037x; 1.0604x over previous
import functools

import jax
import jax.numpy as jnp
import numpy as np
from jax import lax
from jax.experimental import pallas as pl
from jax.experimental.pallas import tpu as pltpu
from jax.experimental.pallas import tpu_sc as plsc

N = 10000
E = 160000
B = 128
EMB = 128
BINS = 512
FIN = 92

F32 = jnp.float32


_BE = 640
_BN = 1000


def _edge_feat_body(gamma_ref, ea_ref, centers_ref, wrbf_ref, brbf_ref,
                    wk0_ref, bk0_ref, wm0_ref, bm0_ref,
                    wk1_ref, bk1_ref, wm1_ref, bm1_ref,
                    we_ref, be_ref,
                    ef0_ref, ef1_ref, efe_ref):
    ea = ea_ref[...]
    nrm2 = ea[0] ** 2 + ea[1] ** 2 + ea[2] ** 2
    s = -0.75 * lax.rsqrt(nrm2)
    gamma = gamma_ref[0]
    d = s[:, None] - centers_ref[...]
    r = jnp.exp(-gamma * d * d)
    ef = jax.nn.softplus(
        jnp.dot(r, wrbf_ref[...], preferred_element_type=F32) + brbf_ref[...])
    dotp = lambda w, b: jnp.dot(ef, w[...], preferred_element_type=F32) + b[...]
    ef0_ref[...] = jnp.concatenate([dotp(wk0_ref, bk0_ref),
                                    dotp(wm0_ref, bm0_ref)], axis=1)
    ef1_ref[...] = jnp.concatenate([dotp(wk1_ref, bk1_ref),
                                    dotp(wm1_ref, bm1_ref)], axis=1)
    efe_ref[...] = dotp(we_ref, be_ref)


def _edge_features(ea_t, centers, gamma, p):
    g = E // _BE
    full = lambda shp: pl.BlockSpec(shp, lambda i: (0,) * len(shp))
    a0, a1, eq = p['att0'], p['att1'], p['equi']
    args = (ea_t, centers.reshape(1, BINS), p['W_rbf'], p['b_rbf'].reshape(1, EMB),
            a0['Wku'][2 * EMB:], a0['bku'].reshape(1, EMB),
            a0['Wmu'][2 * EMB:], a0['bmu'].reshape(1, EMB),
            a1['Wku'][2 * EMB:], a1['bku'].reshape(1, EMB),
            a1['Wmu'][2 * EMB:], a1['bmu'].reshape(1, EMB),
            eq['We1'][2 * EMB:], eq['be1'].reshape(1, EMB))
    in_specs = [pl.BlockSpec((3, _BE), lambda i: (0, i)),
                full((1, BINS)), full((BINS, EMB)), full((1, EMB)),
                full((EMB, EMB)), full((1, EMB)), full((EMB, EMB)), full((1, EMB)),
                full((EMB, EMB)), full((1, EMB)), full((EMB, EMB)), full((1, EMB)),
                full((EMB, EMB)), full((1, EMB))]
    return pl.pallas_call(
        _edge_feat_body,
        grid=(g,),
        in_specs=[pl.BlockSpec(memory_space=pltpu.SMEM)] + in_specs,
        out_specs=[pl.BlockSpec((_BE, 2 * EMB), lambda i: (i, 0)),
                   pl.BlockSpec((_BE, 2 * EMB), lambda i: (i, 0)),
                   pl.BlockSpec((_BE, EMB), lambda i: (i, 0))],
        out_shape=[jax.ShapeDtypeStruct((E, 2 * EMB), F32),
                   jax.ShapeDtypeStruct((E, 2 * EMB), F32),
                   jax.ShapeDtypeStruct((E, EMB), F32)],
    )(gamma.reshape(1), *args)


_SCALE = float(1.0 / np.sqrt(float(EMB)))


def _qkv_tables(nf, cp):
    q = (jnp.dot(nf, cp['Wq'], preferred_element_type=F32) + cp['bq']) * _SCALE
    k = jnp.dot(nf, cp['Wk'], preferred_element_type=F32) + cp['bk']
    v = jnp.dot(nf, cp['Wv'], preferred_element_type=F32) + cp['bv']
    ka = jnp.dot(k, cp['WkuA'], preferred_element_type=F32)
    kb = jnp.dot(k, cp['WkuB'], preferred_element_type=F32)
    va = jnp.dot(v, cp['WmuA'], preferred_element_type=F32)
    vb = jnp.dot(v, cp['WmuB'], preferred_element_type=F32)
    tdst = jnp.concatenate([q, ka, va], axis=1)
    tsrc = jnp.concatenate([kb, vb], axis=1)
    return tdst, tsrc


def _conv_param_args(cp):
    return (cp['Wq'], cp['bq'].reshape(1, EMB), cp['Wk'], cp['bk'].reshape(1, EMB),
            cp['Wv'], cp['bv'].reshape(1, EMB),
            cp['Wku'][:EMB], cp['Wku'][EMB:2 * EMB],
            cp['Wmu'][:EMB], cp['Wmu'][EMB:2 * EMB])


def _conv_param_dict(refs):
    (wq, bq, wk, bk, wv, bv, wkua, wkub, wmua, wmub) = refs
    return dict(Wq=wq[...], bq=bq[...], Wk=wk[...], bk=bk[...],
                Wv=wv[...], bv=bv[...], WkuA=wkua[...], WkuB=wkub[...],
                WmuA=wmua[...], WmuB=wmub[...])


def _tables0_body(x_ref, wa_ref, ba_ref, *rest):
    cp = _conv_param_dict(rest[:10])
    nf_ref, tdst_ref, tsrc_ref = rest[10:]
    nf = jnp.dot(x_ref[...], wa_ref[...], preferred_element_type=F32) + ba_ref[...]
    nf_ref[...] = nf
    tdst, tsrc = _qkv_tables(nf, cp)
    tdst_ref[...] = tdst
    tsrc_ref[...] = tsrc


def _tables0(x, p):
    g = N // _BN
    full = lambda shp: pl.BlockSpec(shp, lambda i: (0,) * len(shp))
    cp_args = _conv_param_args(p['att0'])
    in_specs = ([pl.BlockSpec((_BN, FIN), lambda i: (i, 0)),
                 full((FIN, EMB)), full((1, EMB))] +
                [full(a.shape) for a in cp_args])
    return pl.pallas_call(
        _tables0_body,
        grid=(g,),
        in_specs=in_specs,
        out_specs=[pl.BlockSpec((_BN, EMB), lambda i: (i, 0)),
                   pl.BlockSpec((_BN, 3 * EMB), lambda i: (i, 0)),
                   pl.BlockSpec((_BN, 2 * EMB), lambda i: (i, 0))],
        out_shape=[jax.ShapeDtypeStruct((N, EMB), F32),
                   jax.ShapeDtypeStruct((N, 3 * EMB), F32),
                   jax.ShapeDtypeStruct((N, 2 * EMB), F32)],
    )(x, p['W_atom'], p['b_atom'].reshape(1, EMB), *cp_args)


def _node_update(nf_prev, agg, cnt, wml, bml):
    out = jnp.dot(agg, wml, preferred_element_type=F32) + cnt * bml
    return jax.nn.softplus(nf_prev + out)


def _update_tables_body(nf_ref, a0_ref, a1_ref, c0_ref, c1_ref,
                        wml_ref, bml_ref, *rest):
    cp = _conv_param_dict(rest[:10])
    nf_ref_o, tdst_ref, tsrc_ref = rest[10:]
    s = a0_ref[...] + a1_ref[...]
    cnt = c0_ref[...] + c1_ref[...]
    nf = _node_update(nf_ref[...], s, cnt, wml_ref[...], bml_ref[...])
    nf_ref_o[...] = nf
    tdst, tsrc = _qkv_tables(nf, cp)
    tdst_ref[...] = tdst
    tsrc_ref[...] = tsrc


def _update_tables(nf, agg, cnt2, conv_prev, conv_next):
    g = N // _BN
    full = lambda shp: pl.BlockSpec(shp, lambda i: (0,) * len(shp))
    cp_args = _conv_param_args(conv_next)
    row = lambda w: pl.BlockSpec((_BN, w), lambda i: (i, 0))
    in_specs = ([row(EMB), row(EMB), row(EMB), row(1), row(1),
                 full((EMB, EMB)), full((1, EMB))] +
                [full(a.shape) for a in cp_args])
    return pl.pallas_call(
        _update_tables_body,
        grid=(g,),
        in_specs=in_specs,
        out_specs=[row(EMB), row(3 * EMB), row(2 * EMB)],
        out_shape=[jax.ShapeDtypeStruct((N, EMB), F32),
                   jax.ShapeDtypeStruct((N, 3 * EMB), F32),
                   jax.ShapeDtypeStruct((N, 2 * EMB), F32)],
    )(nf, agg[0], agg[1], cnt2[0], cnt2[1],
      conv_prev['Wml'], conv_prev['bml'].reshape(1, EMB), *cp_args)


def _equi_tables_body(nf_ref, a0_ref, a1_ref, c0_ref, c1_ref,
                      wml_ref, bml_ref, wes_ref, wed_ref,
                      nf_ref_o, tsrc_ref, tdst_ref):
    cnt = c0_ref[...] + c1_ref[...]
    nf = _node_update(nf_ref[...], a0_ref[...] + a1_ref[...], cnt,
                      wml_ref[...], bml_ref[...])
    nf_ref_o[...] = nf
    tsrc_ref[...] = jnp.dot(nf, wes_ref[...], preferred_element_type=F32)
    tdst_ref[...] = jnp.dot(nf, wed_ref[...], preferred_element_type=F32)


def _equi_tables(nf, agg, cnt2, conv_prev, ep):
    g = N // _BN
    full = lambda shp: pl.BlockSpec(shp, lambda i: (0,) * len(shp))
    row = lambda w: pl.BlockSpec((_BN, w), lambda i: (i, 0))
    in_specs = [row(EMB), row(EMB), row(EMB), row(1), row(1),
                full((EMB, EMB)), full((1, EMB)),
                full((EMB, EMB)), full((EMB, EMB))]
    return pl.pallas_call(
        _equi_tables_body,
        grid=(g,),
        in_specs=in_specs,
        out_specs=[row(EMB), row(EMB), row(EMB)],
        out_shape=[jax.ShapeDtypeStruct((N, EMB), F32),
                   jax.ShapeDtypeStruct((N, EMB), F32),
                   jax.ShapeDtypeStruct((N, EMB), F32)],
    )(nf, agg[0], agg[1], cnt2[0], cnt2[1],
      conv_prev['Wml'], conv_prev['bml'].reshape(1, EMB),
      ep['We1'][:EMB], ep['We1'][EMB:2 * EMB])


def _equi_pool_body(nf_ref, a0_ref, a1_ref, c0_ref, c1_ref, batch_ref,
                    we2_ref, be2_ref, csum_ref, ccnt_ref):
    cnt = jnp.maximum(c0_ref[...] + c1_ref[...], 1.0)
    agg = (a0_ref[...] + a1_ref[...]) / cnt
    nf3 = jax.nn.softplus(
        nf_ref[...] + jnp.dot(agg, we2_ref[...], preferred_element_type=F32)
        + be2_ref[...])
    gid = lax.broadcasted_iota(jnp.int32, (1, B), 1)
    oh = (batch_ref[...] == gid).astype(F32)
    contrib = lax.dot_general(oh, nf3, (((0,), (0,)), ((), ())),
                              preferred_element_type=F32)
    ones = jnp.ones((nf3.shape[0], 1), F32)
    ccontrib = lax.dot_general(oh, ones, (((0,), (0,)), ((), ())),
                               preferred_element_type=F32)

    @pl.when(pl.program_id(0) == 0)
    def _init():
        csum_ref[...] = contrib
        ccnt_ref[...] = ccontrib

    @pl.when(pl.program_id(0) != 0)
    def _acc():
        csum_ref[...] += contrib
        ccnt_ref[...] += ccontrib


def _equi_pool(nf, agg, cnt2, batch2d, ep):
    g = N // _BN
    full = lambda shp: pl.BlockSpec(shp, lambda i: (0,) * len(shp))
    row = lambda w: pl.BlockSpec((_BN, w), lambda i: (i, 0))
    in_specs = [row(EMB), row(EMB), row(EMB), row(1), row(1), row(1),
                full((EMB, EMB)), full((1, EMB))]
    return pl.pallas_call(
        _equi_pool_body,
        grid=(g,),
        in_specs=in_specs,
        out_specs=[full((B, EMB)), full((B, 1))],
        out_shape=[jax.ShapeDtypeStruct((B, EMB), F32),
                   jax.ShapeDtypeStruct((B, 1), F32)],
    )(nf, agg[0], agg[1], cnt2[0], cnt2[1], batch2d,
      ep['We2'], ep['be2'].reshape(1, EMB))


_PAIRS = [(j, k) for j in range(9) for k in range(j + 1, 9)]


def _head_body(csum_ref, ccnt_ref, fm_ref, wout_ref, bout_ref, eq_ref, out_ref):
    crystal = csum_ref[...] / jnp.maximum(ccnt_ref[...], 1.0)
    fm = fm_ref[...]
    crystal2 = jnp.sum(fm * crystal[:, None, :], axis=2)
    out9 = (jnp.dot(crystal2, wout_ref[...], preferred_element_type=F32)
            + bout_ref[...])
    cols = [out9[:, j:j + 1] for j in range(9)]
    for (j, k) in _PAIRS:
        m = eq_ref[:, 9 * j + k:9 * j + k + 1] != 0
        avg = 0.5 * (cols[j] + cols[k])
        cols[j] = jnp.where(m, avg, cols[j])
        cols[k] = jnp.where(m, avg, cols[k])
    out_ref[...] = jnp.concatenate(cols, axis=1)


def _head(csum, ccnt, feat_mask, wout, bout, eqflat):
    full = lambda shp: pl.BlockSpec(shp, lambda: (0,) * len(shp))
    return pl.pallas_call(
        _head_body,
        in_specs=[full((B, EMB)), full((B, 1)), full((B, EMB, EMB)),
                  full((EMB, 9)), full((1, 9)), full((B, 81))],
        out_specs=full((B, 9)),
        out_shape=jax.ShapeDtypeStruct((B, 9), F32),
    )(csum, ccnt, feat_mask, wout, bout.reshape(1, 9), eqflat)


_NC = 2
_NS = 16
_NW = _NC * _NS
_C = 40
_EPT = E // _NW
_NCHUNK = _EPT // _C
_NACC = 10240
_RPT = _NACC // _NS
_HR = _NACC // 128

def _sc_mesh():
    return plsc.VectorSubcoreMesh(core_axis_name="c", subcore_axis_name="s",
                                  num_cores=_NC, num_subcores=_NS)


def _vec_rsqrt(t):
    i = lax.bitcast_convert_type(t, jnp.int32)
    y = lax.bitcast_convert_type(0x5F3759DF - (i >> 1), F32)
    for _ in range(3):
        y = y * (1.5 - 0.5 * t * y * y)
    return y


def _sigmoid(x):
    return 1.0 / (1.0 + jnp.exp(-x))


def _zero_vmem(ref, rows, width):
    def body(i, c):
        for f in range(width // 16):
            ref[i, pl.ds(f * 16, 16)] = jnp.zeros((16,), F32)
        return c
    lax.fori_loop(0, rows, body, 0)


def _copy_zero_to_shared(zbuf, shared, row0, nrows, bufrows):
    r = 0
    while r < nrows:
        sz = min(bufrows, nrows - r)
        pltpu.sync_copy(zbuf.at[pl.ds(0, sz)], shared.at[pl.ds(row0 + r, sz)])
        r += sz


def _conv_edge_body(tdst_hbm, tsrc_hbm, ef_hbm, src_hbm, dst_hbm,
                    agg_hbm, idx_s, idx_d, rows_d, rows_s, ef_b, msg,
                    acc_sh, sem1, sem2, sem3):
    core = lax.axis_index("c")
    sub = lax.axis_index("s")
    wid = core * _NS + sub
    base = wid * _EPT

    _zero_vmem(msg, _C, EMB)
    _copy_zero_to_shared(msg, acc_sh, sub * _RPT, _RPT, _C)
    plsc.subcore_barrier()

    def chunk_body(i, carry):
        off = pl.multiple_of(base + i * _C, 8)
        pltpu.sync_copy(dst_hbm.at[pl.ds(off, _C)], idx_d)
        pltpu.sync_copy(src_hbm.at[pl.ds(off, _C)], idx_s)
        cp1 = pltpu.async_copy(tdst_hbm.at[idx_d], rows_d, sem1)
        cp2 = pltpu.async_copy(tsrc_hbm.at[idx_s], rows_s, sem2)
        cp3 = pltpu.async_copy(ef_hbm.at[pl.ds(off, _C)], ef_b, sem3)
        cp1.wait()
        cp2.wait()
        cp3.wait()

        lanes = lax.iota(jnp.int32, 16)

        def edge_body(e, c):
            alpha = []
            vsum = jnp.zeros((16,), F32)
            vsq = jnp.zeros((16,), F32)
            for f in range(8):
                q = rows_d[e, pl.ds(f * 16, 16)]
                ka = rows_d[e, pl.ds(EMB + f * 16, 16)]
                kb = rows_s[e, pl.ds(f * 16, 16)]
                efk = ef_b[e, pl.ds(f * 16, 16)]
                a = q * (ka + kb + efk)
                alpha.append(a)
                vsum = vsum + a
                vsq = vsq + a * a
            for kk in (8, 4, 2, 1):
                vsum = vsum + vsum[lanes ^ kk]
                vsq = vsq + vsq[lanes ^ kk]
            mean = vsum * (1.0 / EMB)
            var = vsq * (1.0 / EMB) - mean * mean
            rstd = _vec_rsqrt(var + 1e-5)
            for f in range(8):
                g = _sigmoid((alpha[f] - mean) * rstd)
                va = rows_d[e, pl.ds(2 * EMB + f * 16, 16)]
                vb = rows_s[e, pl.ds(EMB + f * 16, 16)]
                efm = ef_b[e, pl.ds(EMB + f * 16, 16)]
                msg[e, pl.ds(f * 16, 16)] = (va + vb + efm) * g
            return c

        lax.fori_loop(0, _C, edge_body, 0)
        pltpu.sync_copy(msg, acc_sh.at[idx_d], add=True)
        return carry

    lax.fori_loop(0, _NCHUNK, chunk_body, 0)
    plsc.subcore_barrier()

    r0 = sub * _RPT
    pltpu.sync_copy(acc_sh.at[pl.ds(r0, _RPT)], agg_hbm.at[core, pl.ds(r0, _RPT)])


def _make_conv_edge():
    out_type = [jax.ShapeDtypeStruct((_NC, _NACC, EMB), F32)]
    scratch = [pltpu.VMEM((_C,), jnp.int32), pltpu.VMEM((_C,), jnp.int32),
               pltpu.VMEM((_C, 3 * EMB), F32), pltpu.VMEM((_C, 2 * EMB), F32),
               pltpu.VMEM((_C, 2 * EMB), F32), pltpu.VMEM((_C, EMB), F32),
               pltpu.VMEM_SHARED((_NACC, EMB), F32)]
    scratch += [pltpu.SemaphoreType.DMA] * 3
    return pl.kernel(_conv_edge_body,
                     out_type=out_type, mesh=_sc_mesh(), scratch_types=scratch)


_CC = 40
_NCHUNK_C = _EPT // _CC


def _count_edge_body(dst_hbm, cnt_hbm, idx_d, ones_b, zbuf, acc_sh, sem1):
    core = lax.axis_index("c")
    sub = lax.axis_index("s")
    wid = core * _NS + sub
    base = wid * _EPT

    _zero_vmem(zbuf, _CC, EMB)
    _zero_vmem(ones_b, _CC, EMB)
    _copy_zero_to_shared(zbuf, acc_sh, sub * _RPT, _RPT, _CC)
    lane0 = jnp.where(lax.iota(jnp.int32, 16) == 0,
                      jnp.float32(1), jnp.float32(0))

    def fill_ones(i, c):
        ones_b[i, pl.ds(0, 16)] = lane0
        return c
    lax.fori_loop(0, _CC, fill_ones, 0)
    plsc.subcore_barrier()

    def chunk_body(i, carry):
        off = pl.multiple_of(base + i * _CC, 8)
        pltpu.sync_copy(dst_hbm.at[pl.ds(off, _CC)], idx_d)
        pltpu.sync_copy(ones_b, acc_sh.at[idx_d], add=True)
        return carry

    lax.fori_loop(0, _NCHUNK_C, chunk_body, 0)
    plsc.subcore_barrier()
    r0 = sub * _RPT
    pltpu.sync_copy(acc_sh.at[pl.ds(r0, _RPT)], cnt_hbm.at[core, pl.ds(r0, _RPT)])


def _make_count_edge():
    scratch = [pltpu.VMEM((_CC,), jnp.int32), pltpu.VMEM((_CC, EMB), F32),
               pltpu.VMEM((_CC, EMB), F32),
               pltpu.VMEM_SHARED((_NACC, EMB), F32),
               pltpu.SemaphoreType.DMA]
    return pl.kernel(_count_edge_body,
                     out_type=[jax.ShapeDtypeStruct((_NC, _NACC, EMB), F32)],
                     mesh=_sc_mesh(), scratch_types=scratch)


def _equi_edge_body(tsrc_hbm, tdst_hbm, ef_hbm, src_hbm, dst_hbm,
                    agg_hbm, idx_s, idx_d, rows_s, rows_d, ef_b, msg,
                    acc_sh, sem1, sem2, sem3):
    core = lax.axis_index("c")
    sub = lax.axis_index("s")
    wid = core * _NS + sub
    base = wid * _EPT

    _zero_vmem(msg, _C, EMB)
    _copy_zero_to_shared(msg, acc_sh, sub * _RPT, _RPT, _C)
    plsc.subcore_barrier()

    def chunk_body(i, carry):
        off = pl.multiple_of(base + i * _C, 8)
        pltpu.sync_copy(dst_hbm.at[pl.ds(off, _C)], idx_d)
        pltpu.sync_copy(src_hbm.at[pl.ds(off, _C)], idx_s)
        cp1 = pltpu.async_copy(tdst_hbm.at[idx_d], rows_d, sem1)
        cp2 = pltpu.async_copy(tsrc_hbm.at[idx_s], rows_s, sem2)
        cp3 = pltpu.async_copy(ef_hbm.at[pl.ds(off, _C)], ef_b, sem3)
        cp1.wait()
        cp2.wait()
        cp3.wait()

        def edge_body(e, c):
            for f in range(8):
                t = (rows_s[e, pl.ds(f * 16, 16)]
                     + rows_d[e, pl.ds(f * 16, 16)]
                     + ef_b[e, pl.ds(f * 16, 16)])
                msg[e, pl.ds(f * 16, 16)] = t * _sigmoid(t)
            return c

        lax.fori_loop(0, _C, edge_body, 0)
        pltpu.sync_copy(msg, acc_sh.at[idx_d], add=True)
        return carry

    lax.fori_loop(0, _NCHUNK, chunk_body, 0)
    plsc.subcore_barrier()
    r0 = sub * _RPT
    pltpu.sync_copy(acc_sh.at[pl.ds(r0, _RPT)], agg_hbm.at[core, pl.ds(r0, _RPT)])


def _make_equi_edge():
    scratch = [pltpu.VMEM((_C,), jnp.int32), pltpu.VMEM((_C,), jnp.int32),
               pltpu.VMEM((_C, EMB), F32), pltpu.VMEM((_C, EMB), F32),
               pltpu.VMEM((_C, EMB), F32), pltpu.VMEM((_C, EMB), F32),
               pltpu.VMEM_SHARED((_NACC, EMB), F32)]
    scratch += [pltpu.SemaphoreType.DMA] * 3
    return pl.kernel(_equi_edge_body,
                     out_type=[jax.ShapeDtypeStruct((_NC, _NACC, EMB), F32)],
                     mesh=_sc_mesh(), scratch_types=scratch)


def kernel(x, edge_attr, feat_mask, params, edge_index, batch, equality):
    p = params
    src = edge_index[0].astype(jnp.int32)
    dst = edge_index[1].astype(jnp.int32)
    ea_t = edge_attr.T
    centers = jnp.linspace(-4.0, 0.0, BINS)
    gamma = 1.0 / (centers[1] - centers[0]) ** 2

    ef0, ef1, efe = _edge_features(ea_t, centers, gamma, p)
    nf0, td0, ts0 = _tables0(x, p)
    cnts = _make_count_edge()(dst)
    if isinstance(cnts, (tuple, list)):
        cnts = cnts[0]
    cnt2 = cnts[:, :, 0:1]
    agg0 = _make_conv_edge()(td0, ts0, ef0, src, dst)
    if isinstance(agg0, (tuple, list)):
        agg0 = agg0[0]
    nf1, td1, ts1 = _update_tables(nf0, agg0, cnt2, p['att0'], p['att1'])
    agg1 = _make_conv_edge()(td1, ts1, ef1, src, dst)
    if isinstance(agg1, (tuple, list)):
        agg1 = agg1[0]
    nf2, tse, tde = _equi_tables(nf1, agg1, cnt2, p['att1'], p['equi'])
    agge = _make_equi_edge()(tse, tde, efe, src, dst)
    if isinstance(agge, (tuple, list)):
        agge = agge[0]
    csum, ccnt = _equi_pool(nf2, agge, cnt2, batch.astype(jnp.int32).reshape(N, 1),
                            p['equi'])
    eqflat = equality[:, :9, :9].reshape(B, 81).astype(jnp.int32)
    out9 = _head(csum, ccnt, feat_mask, p['W_out'], p['b_out'], eqflat)
    return out9.reshape(B, 3, 3)
